# Initial kernel scaffold; baseline (speedup 1.0000x reference)
import jax
import jax.numpy as jnp
from jax.experimental import pallas as pl


def kernel(joint_input, x_target, W, b, centroids):
    raise NotImplementedError("write your pallas kernel here")


# speedup vs baseline: 3.2486x; 3.2486x over previous
import jax
import jax.numpy as jnp
from jax.experimental import pallas as pl
from jax.experimental.pallas import tpu as pltpu

_N, _T, _U, _J = 8, 128, 8, 512
_K, _D = 4096, 320

_QT = 128
_G = 32
_R = _G * _U


def _knn_body(q_ref, ct_ref, idx_ref):
    ct = ct_ref[...]
    csq = jnp.sum(ct * ct, axis=0, keepdims=True)
    dists = csq - 2.0 * jnp.dot(q_ref[...], ct,
                                preferred_element_type=jnp.float32)
    m = jnp.min(dists, axis=1, keepdims=True)
    kio = jax.lax.broadcasted_iota(jnp.int32, dists.shape, 1)
    idx = jnp.min(jnp.where(dists == m, kio, _K), axis=1)
    idx_ref[0, 0, :] = idx


def _main_body(x_ref, wt_ref, b_ref, idx_ref, out_ref):
    logits = jnp.dot(x_ref[...], wt_ref[...],
                     preferred_element_type=jnp.float32) + b_ref[...]
    m = jnp.max(logits, axis=1)
    s = jnp.sum(jnp.exp(logits - m[:, None]), axis=1)
    lse = m + jnp.log(s)
    idx = idx_ref[0, 0, :]
    kio = jax.lax.broadcasted_iota(jnp.int32, logits.shape, 1)
    sel = jnp.sum(jnp.where(kio == idx[:, None], logits, 0.0), axis=1)
    out_ref[0, 0, :] = sel - lse


def kernel(joint_input, x_target, W, b, centroids):
    n, t, u, j = joint_input.shape
    k, d = centroids.shape
    x = joint_input.reshape(n * t * u, j)
    q = x_target.reshape(n * t, d)
    ct = centroids.T
    wt = W.T

    nq_tiles = (n * t) // _QT
    idx = pl.pallas_call(
        _knn_body,
        grid=(nq_tiles,),
        in_specs=[
            pl.BlockSpec((_QT, d), lambda i: (i, 0)),
            pl.BlockSpec((d, k), lambda i: (0, 0)),
        ],
        out_specs=pl.BlockSpec((1, 1, _QT), lambda i: (i, 0, 0)),
        out_shape=jax.ShapeDtypeStruct((nq_tiles, 1, _QT), jnp.int32),
        compiler_params=pltpu.CompilerParams(
            dimension_semantics=("parallel",)),
    )(q, ct)

    idx_rows = jnp.repeat(idx.reshape(n * t), u)
    nr_tiles = (n * t * u) // _R
    idx3 = idx_rows.reshape(nr_tiles, 1, _R)

    out = pl.pallas_call(
        _main_body,
        grid=(nr_tiles,),
        in_specs=[
            pl.BlockSpec((_R, j), lambda i: (i, 0)),
            pl.BlockSpec((j, k), lambda i: (0, 0)),
            pl.BlockSpec((1, k), lambda i: (0, 0)),
            pl.BlockSpec((1, 1, _R), lambda i: (i, 0, 0)),
        ],
        out_specs=pl.BlockSpec((1, 1, _R), lambda i: (i, 0, 0)),
        out_shape=jax.ShapeDtypeStruct((nr_tiles, 1, _R), jnp.float32),
        compiler_params=pltpu.CompilerParams(
            dimension_semantics=("parallel",)),
    )(x, wt, b.reshape(1, k), idx3)

    return out.reshape(n, t, u)


# speedup vs baseline: 3.3756x; 1.0391x over previous
import jax
import jax.numpy as jnp
from jax.experimental import pallas as pl
from jax.experimental.pallas import tpu as pltpu

_N, _T, _U, _J = 8, 128, 8, 512
_K, _D = 4096, 320

_QT = 128
_G = 32
_R = _G * _U


def _knn_body(q_ref, ct_ref, idx_ref):
    ct = ct_ref[...]
    csq = jnp.sum(ct * ct, axis=0, keepdims=True)
    dists = csq - 2.0 * jnp.dot(q_ref[...], ct,
                                preferred_element_type=jnp.float32)
    m = jnp.min(dists, axis=1, keepdims=True)
    kio = jax.lax.broadcasted_iota(jnp.int32, dists.shape, 1)
    idx = jnp.min(jnp.where(dists == m, kio, _K), axis=1)
    idx_ref[0, 0, :] = idx


def _main_body(x_ref, wt_ref, b_ref, idx_ref, out_ref):
    logits = jnp.dot(x_ref[...], wt_ref[...],
                     preferred_element_type=jnp.float32) + b_ref[...]
    m = jnp.max(logits, axis=1)
    s = jnp.sum(jnp.exp(logits - m[:, None]), axis=1)
    lse = m + jnp.log(s)
    idx = idx_ref[0, 0, :]
    kio = jax.lax.broadcasted_iota(jnp.int32, logits.shape, 1)
    sel = jnp.sum(jnp.where(kio == idx[:, None], logits, 0.0), axis=1)
    out_ref[0, 0, :] = sel - lse


def kernel(joint_input, x_target, W, b, centroids):
    n, t, u, j = joint_input.shape
    k, d = centroids.shape
    x = joint_input.reshape(n * t * u, j).astype(jnp.bfloat16)
    q = x_target.reshape(n * t, d)
    ct = centroids.T
    wt = W.T.astype(jnp.bfloat16)

    nq_tiles = (n * t) // _QT
    idx = pl.pallas_call(
        _knn_body,
        grid=(nq_tiles,),
        in_specs=[
            pl.BlockSpec((_QT, d), lambda i: (i, 0)),
            pl.BlockSpec((d, k), lambda i: (0, 0)),
        ],
        out_specs=pl.BlockSpec((1, 1, _QT), lambda i: (i, 0, 0)),
        out_shape=jax.ShapeDtypeStruct((nq_tiles, 1, _QT), jnp.int32),
        compiler_params=pltpu.CompilerParams(
            dimension_semantics=("parallel",)),
    )(q, ct)

    idx_rows = jnp.repeat(idx.reshape(n * t), u)
    nr_tiles = (n * t * u) // _R
    idx3 = idx_rows.reshape(nr_tiles, 1, _R)

    out = pl.pallas_call(
        _main_body,
        grid=(nr_tiles,),
        in_specs=[
            pl.BlockSpec((_R, j), lambda i: (i, 0)),
            pl.BlockSpec((j, k), lambda i: (0, 0)),
            pl.BlockSpec((1, k), lambda i: (0, 0)),
            pl.BlockSpec((1, 1, _R), lambda i: (i, 0, 0)),
        ],
        out_specs=pl.BlockSpec((1, 1, _R), lambda i: (i, 0, 0)),
        out_shape=jax.ShapeDtypeStruct((nr_tiles, 1, _R), jnp.float32),
        compiler_params=pltpu.CompilerParams(
            dimension_semantics=("parallel",)),
    )(x, wt, b.reshape(1, k), idx3)

    return out.reshape(n, t, u)
